# baseline (speedup 1.0000x reference)
import functools

import jax
import jax.numpy as jnp
from jax import lax
from jax.experimental import pallas as pl
from jax.experimental.pallas import tpu as pltpu
from jax.experimental.pallas import tpu_sc as plsc

_IDX_MINOR = 128
_NW = 32


def _rowmean_body(x_ref, o_ref):
    inv = jnp.float32(1.0 / x_ref.shape[0])
    o_ref[...] = jnp.sum(x_ref[...], axis=0, keepdims=True) * inv


def _rowmean(tT, vt=8192):
    D, V = tT.shape
    grid = (V + vt - 1) // vt
    return pl.pallas_call(
        _rowmean_body,
        grid=(grid,),
        in_specs=[pl.BlockSpec((D, vt), lambda i: (0, i))],
        out_specs=pl.BlockSpec((1, vt), lambda i: (0, i)),
        out_shape=jax.ShapeDtypeStruct((1, V), jnp.float32),
    )(tT)


def _gather_body(n_per_w, rmean_hbm, ctx_hbm, out_hbm, idx_v, out_v, sem):
    rows = n_per_w // _IDX_MINOR
    wid = lax.axis_index("s") * 2 + lax.axis_index("c")
    pltpu.sync_copy(ctx_hbm.at[pl.ds(wid * rows, rows)], idx_v)
    copies = [
        pltpu.async_copy(rmean_hbm.at[idx_v.at[j]],
                         out_v.at[pl.ds(j * _IDX_MINOR, _IDX_MINOR)], sem)
        for j in range(rows)
    ]
    for c in copies:
        c.wait()
    pltpu.sync_copy(out_v, out_hbm.at[pl.ds(wid * n_per_w, n_per_w)])


def _gather(rmean_flat, ctx2d):
    n = ctx2d.shape[0] * ctx2d.shape[1]
    n_per_w = n // _NW
    mesh = plsc.VectorSubcoreMesh(core_axis_name="c", subcore_axis_name="s")
    fn = pl.kernel(
        functools.partial(_gather_body, n_per_w),
        out_type=jax.ShapeDtypeStruct((n,), jnp.float32),
        mesh=mesh,
        scratch_types=[
            pltpu.VMEM((n_per_w // _IDX_MINOR, _IDX_MINOR), jnp.int32),
            pltpu.VMEM((n_per_w,), jnp.float32),
            pltpu.SemaphoreType.DMA,
        ],
    )
    return fn(rmean_flat, ctx2d)


def _proj_body(w_ref, x_ref, b_ref, o_ref):
    acc = lax.dot_general(
        w_ref[...], x_ref[...], (((0,), (0,)), ((), ())),
        preferred_element_type=jnp.float32)
    o_ref[...] = acc + jnp.swapaxes(b_ref[...], 0, 1)


def _project(Wp, xT, b2d, vt=2048):
    D, V = Wp.shape
    B = xT.shape[1]
    grid = (V + vt - 1) // vt
    return pl.pallas_call(
        _proj_body,
        grid=(grid,),
        in_specs=[
            pl.BlockSpec((D, vt), lambda i: (0, i)),
            pl.BlockSpec((D, B), lambda i: (0, 0)),
            pl.BlockSpec((1, vt), lambda i: (0, i)),
        ],
        out_specs=pl.BlockSpec((vt, B), lambda i: (i, 0)),
        out_shape=jax.ShapeDtypeStruct((V, B), jnp.float32),
    )(Wp, xT, b2d)


def kernel(context, table, W, b):
    B, C = context.shape
    V, D = table.shape
    ctxT = jnp.swapaxes(context, 0, 1).astype(jnp.int32)
    ctx2d = ctxT.reshape(-1, _IDX_MINOR)
    tT = jnp.swapaxes(table, 0, 1)
    Wp = jnp.swapaxes(W, 0, 1)
    rm = _rowmean(tT)
    embT = _gather(rm.reshape(V), ctx2d)
    xT = embT.reshape(C, B)
    outT = _project(Wp, xT, b.reshape(1, V))
    return jnp.swapaxes(outT, 0, 1)


# speedup vs baseline: 1.0755x; 1.0755x over previous
import functools

import jax
import jax.numpy as jnp
from jax import lax
from jax.experimental import pallas as pl
from jax.experimental.pallas import tpu as pltpu
from jax.experimental.pallas import tpu_sc as plsc

_IDX_MINOR = 128
_NW = 32


def _rowmean_body(x_ref, o_ref):
    inv = jnp.float32(1.0 / x_ref.shape[0])
    o_ref[...] = jnp.sum(x_ref[...], axis=0) * inv


def _rowmean(tT, vt=25600):
    D, V = tT.shape
    grid = (V + vt - 1) // vt
    return pl.pallas_call(
        _rowmean_body,
        grid=(grid,),
        in_specs=[pl.BlockSpec((D, vt), lambda i: (0, i))],
        out_specs=pl.BlockSpec((vt,), lambda i: (i,)),
        out_shape=jax.ShapeDtypeStruct((V,), jnp.float32),
    )(tT)


def _gather_body(rows_per_w, rmean_hbm, ctx_hbm, out_hbm, idx_v, out_v, sem):
    ncol = ctx_hbm.shape[1] // _IDX_MINOR
    wid = lax.axis_index("s") * 2 + lax.axis_index("c")
    pltpu.sync_copy(ctx_hbm.at[pl.ds(wid * rows_per_w, rows_per_w)], idx_v)
    copies = [
        pltpu.async_copy(
            rmean_hbm.at[idx_v.at[j // ncol, pl.ds((j % ncol) * _IDX_MINOR,
                                                   _IDX_MINOR)]],
            out_v.at[j // ncol, pl.ds((j % ncol) * _IDX_MINOR, _IDX_MINOR)],
            sem)
        for j in range(rows_per_w * ncol)
    ]
    for c in copies:
        c.wait()
    pltpu.sync_copy(out_v, out_hbm.at[pl.ds(wid * rows_per_w, rows_per_w)])


def _gather(rmean_flat, ctxT):
    C, B = ctxT.shape
    rows_per_w = C // _NW
    mesh = plsc.VectorSubcoreMesh(core_axis_name="c", subcore_axis_name="s")
    fn = pl.kernel(
        functools.partial(_gather_body, rows_per_w),
        out_type=jax.ShapeDtypeStruct((C, B), jnp.float32),
        mesh=mesh,
        scratch_types=[
            pltpu.VMEM((rows_per_w, B), jnp.int32),
            pltpu.VMEM((rows_per_w, B), jnp.float32),
            pltpu.SemaphoreType.DMA,
        ],
    )
    return fn(rmean_flat, ctxT)


def _proj_body(w_ref, x_ref, b_ref, o_ref):
    acc = lax.dot_general(
        w_ref[...], x_ref[...], (((0,), (0,)), ((), ())),
        preferred_element_type=jnp.float32)
    o_ref[...] = acc + jnp.swapaxes(b_ref[...], 0, 1)


def _project(Wp, xT, b2d, vt=4096):
    D, V = Wp.shape
    B = xT.shape[1]
    grid = (V + vt - 1) // vt
    return pl.pallas_call(
        _proj_body,
        grid=(grid,),
        in_specs=[
            pl.BlockSpec((D, vt), lambda i: (0, i)),
            pl.BlockSpec((D, B), lambda i: (0, 0)),
            pl.BlockSpec((1, vt), lambda i: (0, i)),
        ],
        out_specs=pl.BlockSpec((vt, B), lambda i: (i, 0)),
        out_shape=jax.ShapeDtypeStruct((V, B), jnp.float32),
    )(Wp, xT, b2d)


def kernel(context, table, W, b):
    B, C = context.shape
    V, D = table.shape
    ctxT = jnp.swapaxes(context, 0, 1).astype(jnp.int32)
    tT = jnp.swapaxes(table, 0, 1)
    Wp = jnp.swapaxes(W, 0, 1)
    rm = _rowmean(tT)
    xT = _gather(rm, ctxT)
    outT = _project(Wp, xT, b.reshape(1, V))
    return jnp.swapaxes(outT, 0, 1)
